# baseline (speedup 1.0000x reference)
import functools
import math

import jax
import jax.numpy as jnp
from jax.experimental import pallas as pl
from jax.experimental.pallas import tpu as pltpu


def _pow12(z):
    return jnp.exp2(1.2 * jnp.log2(z))


def _gen_kernel(x_ref, w1t_ref, b1_ref, w2pt_ref, b2p_ref,
                v1t_ref, c1_ref, v2pt_ref, c2p_ref,
                blk_ref, v_ref, *, bd, h):
    xb = x_ref[...].astype(jnp.bfloat16)
    h1 = jax.nn.relu(
        jnp.dot(xb, w1t_ref[...], preferred_element_type=jnp.float32)
        + b1_ref[...]).astype(jnp.bfloat16)
    o = (jnp.dot(h1, w2pt_ref[...], preferred_element_type=jnp.float32)
         + b2p_ref[...])

    u = jax.nn.relu(
        jnp.dot(xb, v1t_ref[...], preferred_element_type=jnp.float32)
        + c1_ref[...]).astype(jnp.bfloat16)
    v_ref[...] = (jnp.dot(u, v2pt_ref[...],
                          preferred_element_type=jnp.float32)
                  + c2p_ref[...])

    jh = bd * h
    mean = o[:, 0:jh]
    for i in range(1, bd):
        mean = mean + o[:, i * jh:(i + 1) * jh]
    mean = mean * (1.0 / bd)
    cent = [o[:, i * jh:(i + 1) * jh] - mean for i in range(bd)]
    norm = _pow12(jnp.abs(cent[0]))
    for i in range(1, bd):
        norm = norm + _pow12(jnp.abs(cent[i]))
    norm = jnp.exp2(jnp.log2(norm) * (1.0 / 1.2))
    maxn = norm[:, 0:h]
    for j in range(1, bd):
        maxn = jnp.maximum(maxn, norm[:, j * h:(j + 1) * h])
    rden = 1.0 / jnp.tile(maxn, (1, bd))
    blk_ref[...] = jnp.concatenate([c * rden for c in cent],
                                   axis=1).astype(blk_ref.dtype)


def _scan_kernel(a0_ref, a_ref, v_ref, out_ref, state_ref, *, t_chunk, bd):
    @pl.when(pl.program_id(1) == 0)
    def _init():
        state_ref[...] = jnp.broadcast_to(a0_ref[...], state_ref.shape)

    def step(t, s):
        st = jnp.tile(s, (1, bd, 1))
        m = a_ref[:, t].astype(jnp.float32) * st
        s_new = (m.reshape(2, bd, bd, m.shape[-1]).sum(axis=2)
                 + v_ref[:, t])
        out_ref[:, t] = s_new
        return s_new

    state_ref[...] = jax.lax.fori_loop(0, t_chunk, step, state_ref[...],
                                       unroll=8)


def kernel(x, W1, b1, W2, b2, V1, c1, V2, c2, a0):
    bs, seq, emb = x.shape
    _, h, bd = a0.shape
    n = emb * bd
    rows = bs * seq
    bf16 = jnp.bfloat16

    w2pt = (W2.astype(bf16).reshape(h, bd, bd, n)
            .transpose(3, 1, 2, 0).reshape(n, n))
    b2p = b2.reshape(h, bd, bd).transpose(1, 2, 0).reshape(1, n)
    v2pt = (V2.astype(bf16).reshape(h, bd, emb)
            .transpose(2, 1, 0).reshape(emb, emb))
    c2p = c2.reshape(h, bd).T.reshape(1, emb)
    w1t = W1.astype(bf16).transpose(1, 0)
    v1t = V1.astype(bf16).transpose(1, 0)
    a0p = a0.reshape(h, bd).T
    xf = x.reshape(rows, emb)

    m_tile = 256
    n_m = rows // m_tile

    gen = pl.pallas_call(
        functools.partial(_gen_kernel, bd=bd, h=h),
        grid=(n_m,),
        in_specs=[
            pl.BlockSpec((m_tile, emb), lambda m: (m, 0)),
            pl.BlockSpec((emb, n), lambda m: (0, 0)),
            pl.BlockSpec((1, n), lambda m: (0, 0)),
            pl.BlockSpec((n, n), lambda m: (0, 0)),
            pl.BlockSpec((1, n), lambda m: (0, 0)),
            pl.BlockSpec((emb, emb), lambda m: (0, 0)),
            pl.BlockSpec((1, emb), lambda m: (0, 0)),
            pl.BlockSpec((emb, emb), lambda m: (0, 0)),
            pl.BlockSpec((1, emb), lambda m: (0, 0)),
        ],
        out_specs=[
            pl.BlockSpec((m_tile, n), lambda m: (m, 0)),
            pl.BlockSpec((m_tile, emb), lambda m: (m, 0)),
        ],
        out_shape=[
            jax.ShapeDtypeStruct((rows, n), jnp.bfloat16),
            jax.ShapeDtypeStruct((rows, emb), jnp.float32),
        ],
        compiler_params=pltpu.CompilerParams(
            dimension_semantics=("parallel",),
            vmem_limit_bytes=57 * 1024 * 1024,
        ),
    )
    blk, v = gen(xf, w1t, b1.reshape(1, n), w2pt, b2p,
                 v1t, c1.reshape(1, emb), v2pt, c2p)

    a_seq = blk.reshape(bs, seq, bd * bd, h)
    v_seq = v.reshape(bs, seq, bd, h)

    t_chunk = 128
    n_chunks = seq // t_chunk
    scan = pl.pallas_call(
        functools.partial(_scan_kernel, t_chunk=t_chunk, bd=bd),
        grid=(bs // 2, n_chunks),
        in_specs=[
            pl.BlockSpec((1, bd, h), lambda b, c: (0, 0, 0)),
            pl.BlockSpec((2, t_chunk, bd * bd, h), lambda b, c: (b, c, 0, 0)),
            pl.BlockSpec((2, t_chunk, bd, h), lambda b, c: (b, c, 0, 0)),
        ],
        out_specs=pl.BlockSpec((2, t_chunk, bd, h), lambda b, c: (b, c, 0, 0)),
        out_shape=jax.ShapeDtypeStruct((bs, seq, bd, h), jnp.float32),
        scratch_shapes=[pltpu.VMEM((2, bd, h), jnp.float32)],
        compiler_params=pltpu.CompilerParams(
            dimension_semantics=("parallel", "arbitrary"),
            vmem_limit_bytes=64 * 1024 * 1024,
        ),
    )
    states = scan(a0p.reshape(1, bd, h), a_seq, v_seq)
    return states.transpose(0, 1, 3, 2).reshape(bs, seq, emb)


# speedup vs baseline: 1.0207x; 1.0207x over previous
import functools
import math

import jax
import jax.numpy as jnp
from jax.experimental import pallas as pl
from jax.experimental.pallas import tpu as pltpu


def _pow12(z):
    return jnp.exp2(1.2 * jnp.log2(z))


def _gen_kernel(x_ref, w1t_ref, b1_ref, w2pt_ref, b2p_ref,
                v1t_ref, c1_ref, v2pt_ref, c2p_ref,
                blk_ref, v_ref, *, bd, h):
    xb = x_ref[...]
    h1 = jax.nn.relu(
        jnp.dot(xb, w1t_ref[...], preferred_element_type=jnp.float32)
        + b1_ref[...]).astype(jnp.bfloat16)
    o = (jnp.dot(h1, w2pt_ref[...], preferred_element_type=jnp.float32)
         + b2p_ref[...])

    u = jax.nn.relu(
        jnp.dot(xb, v1t_ref[...], preferred_element_type=jnp.float32)
        + c1_ref[...]).astype(jnp.bfloat16)
    v_ref[...] = (jnp.dot(u, v2pt_ref[...],
                          preferred_element_type=jnp.float32)
                  + c2p_ref[...])

    jh = bd * h
    mean = o[:, 0:jh]
    for i in range(1, bd):
        mean = mean + o[:, i * jh:(i + 1) * jh]
    mean = mean * (1.0 / bd)
    cent = [o[:, i * jh:(i + 1) * jh] - mean for i in range(bd)]
    norm = _pow12(jnp.abs(cent[0]))
    for i in range(1, bd):
        norm = norm + _pow12(jnp.abs(cent[i]))
    norm = jnp.exp2(jnp.log2(norm) * (1.0 / 1.2))
    maxn = norm[:, 0:h]
    for j in range(1, bd):
        maxn = jnp.maximum(maxn, norm[:, j * h:(j + 1) * h])
    rden = 1.0 / jnp.tile(maxn, (1, bd))
    blk_ref[...] = jnp.concatenate([c * rden for c in cent],
                                   axis=1).astype(blk_ref.dtype)


def _scan_kernel(a0_ref, a_ref, v_ref, out_ref, state_ref, *, t_chunk, bd):
    @pl.when(pl.program_id(1) == 0)
    def _init():
        state_ref[...] = jnp.broadcast_to(a0_ref[...], state_ref.shape)

    def step(t, s):
        st = jnp.tile(s, (1, bd, 1))
        m = a_ref[:, t].astype(jnp.float32) * st
        s_new = (m.reshape(2, bd, bd, m.shape[-1]).sum(axis=2)
                 + v_ref[:, t])
        out_ref[:, t] = s_new
        return s_new

    state_ref[...] = jax.lax.fori_loop(0, t_chunk, step, state_ref[...],
                                       unroll=8)


def kernel(x, W1, b1, W2, b2, V1, c1, V2, c2, a0):
    bs, seq, emb = x.shape
    _, h, bd = a0.shape
    n = emb * bd
    rows = bs * seq
    bf16 = jnp.bfloat16

    w2pt = W2.reshape(h, bd, bd, n).transpose(1, 2, 0, 3).reshape(n, n).T
    b2p = b2.reshape(h, bd, bd).transpose(1, 2, 0).reshape(1, n)
    v2pt = V2.reshape(h, bd, emb).transpose(1, 0, 2).reshape(emb, emb).T
    c2p = c2.reshape(h, bd).T.reshape(1, emb)
    a0p = a0.reshape(h, bd).T
    xf = x.reshape(rows, emb).astype(bf16)

    m_tile = 512
    n_m = rows // m_tile

    gen = pl.pallas_call(
        functools.partial(_gen_kernel, bd=bd, h=h),
        grid=(n_m,),
        in_specs=[
            pl.BlockSpec((m_tile, emb), lambda m: (m, 0)),
            pl.BlockSpec((emb, n), lambda m: (0, 0)),
            pl.BlockSpec((1, n), lambda m: (0, 0)),
            pl.BlockSpec((n, n), lambda m: (0, 0)),
            pl.BlockSpec((1, n), lambda m: (0, 0)),
            pl.BlockSpec((emb, emb), lambda m: (0, 0)),
            pl.BlockSpec((1, emb), lambda m: (0, 0)),
            pl.BlockSpec((emb, emb), lambda m: (0, 0)),
            pl.BlockSpec((1, emb), lambda m: (0, 0)),
        ],
        out_specs=[
            pl.BlockSpec((m_tile, n), lambda m: (m, 0)),
            pl.BlockSpec((m_tile, emb), lambda m: (m, 0)),
        ],
        out_shape=[
            jax.ShapeDtypeStruct((rows, n), jnp.bfloat16),
            jax.ShapeDtypeStruct((rows, emb), jnp.float32),
        ],
        compiler_params=pltpu.CompilerParams(
            dimension_semantics=("parallel",),
            vmem_limit_bytes=62 * 1024 * 1024,
        ),
    )
    blk, v = gen(xf, W1.T.astype(bf16), b1.reshape(1, n),
                 w2pt.astype(bf16), b2p,
                 V1.T.astype(bf16), c1.reshape(1, emb),
                 v2pt.astype(bf16), c2p)

    a_seq = blk.reshape(bs, seq, bd * bd, h)
    v_seq = v.reshape(bs, seq, bd, h)

    t_chunk = 128
    n_chunks = seq // t_chunk
    scan = pl.pallas_call(
        functools.partial(_scan_kernel, t_chunk=t_chunk, bd=bd),
        grid=(bs // 2, n_chunks),
        in_specs=[
            pl.BlockSpec((1, bd, h), lambda b, c: (0, 0, 0)),
            pl.BlockSpec((2, t_chunk, bd * bd, h), lambda b, c: (b, c, 0, 0)),
            pl.BlockSpec((2, t_chunk, bd, h), lambda b, c: (b, c, 0, 0)),
        ],
        out_specs=pl.BlockSpec((2, t_chunk, bd, h), lambda b, c: (b, c, 0, 0)),
        out_shape=jax.ShapeDtypeStruct((bs, seq, bd, h), jnp.float32),
        scratch_shapes=[pltpu.VMEM((2, bd, h), jnp.float32)],
        compiler_params=pltpu.CompilerParams(
            dimension_semantics=("parallel", "arbitrary"),
            vmem_limit_bytes=64 * 1024 * 1024,
        ),
    )
    states = scan(a0p.reshape(1, bd, h), a_seq, v_seq)
    return states.transpose(0, 1, 3, 2).reshape(bs, seq, emb)
